# Initial kernel scaffold; baseline (speedup 1.0000x reference)
import jax
import jax.numpy as jnp
from jax.experimental import pallas as pl


def kernel(h0, W1, b1, W2, b2, anchor_entail, anchor_contra, anchor_neutral):
    raise NotImplementedError("write your pallas kernel here")


# speedup vs baseline: 1.9359x; 1.9359x over previous
import functools

import jax
import jax.numpy as jnp
from jax.experimental import pallas as pl

_DIM = 1024
_NUM_LAYERS = 6
_STRENGTHS = (0.1, 0.1, 0.05)
_BLOCK = 512


def _collapse_block(h_ref, w1_ref, b1_ref, w2_ref, b2_ref, anch_ref, out_ref):
    h = h_ref[...]
    w1 = w1_ref[...]
    w2 = w2_ref[...]
    b1 = b1_ref[...]
    b2 = b2_ref[...]
    anch = anch_ref[...]
    anorm = jnp.sqrt(jnp.sum(anch * anch, axis=-1, keepdims=True))
    anchors = anch / jnp.maximum(anorm, 1e-12)

    for _ in range(_NUM_LAYERS):
        hnorm = jnp.sqrt(jnp.sum(h * h, axis=-1, keepdims=True))
        h_n = h / jnp.maximum(hnorm, 1e-12)
        hb = h.astype(jnp.bfloat16)
        hidden = jnp.tanh(
            jax.lax.dot_general(hb, w1, (((1,), (1,)), ((), ())),
                                preferred_element_type=jnp.float32) + b1)
        delta = jax.lax.dot_general(hidden.astype(jnp.bfloat16), w2,
                                    (((1,), (1,)), ((), ())),
                                    preferred_element_type=jnp.float32) + b2
        force = jnp.zeros_like(h)
        for i in range(3):
            a = anchors[i][None, :]
            align = jnp.sum(h_n * a, axis=-1, keepdims=True)
            div = 1.0 - align
            diff = h - a
            dnorm = jnp.sqrt(jnp.sum(diff * diff, axis=-1, keepdims=True))
            direction = diff / jnp.maximum(dnorm, 1e-12)
            force = force + _STRENGTHS[i] * div * direction
        h = h + delta - force
        hnorm2 = jnp.sqrt(jnp.sum(h * h, axis=-1, keepdims=True))
        h = jnp.where(hnorm2 > 10.0, h * (10.0 / (hnorm2 + 1e-8)), h)
    out_ref[...] = h


@functools.partial(jax.jit, static_argnames=())
def kernel(h0, W1, b1, W2, b2, anchor_entail, anchor_contra, anchor_neutral):
    h = h0
    if h.ndim == 1:
        h = h[None, :]
    n = h.shape[0]
    anchors = jnp.stack([anchor_entail, anchor_contra, anchor_neutral])
    w1b = W1.astype(jnp.bfloat16)
    w2b = W2.astype(jnp.bfloat16)
    b1r = b1.reshape(1, _DIM)
    b2r = b2.reshape(1, _DIM)
    grid = (n // _BLOCK,)
    out = pl.pallas_call(
        _collapse_block,
        grid=grid,
        in_specs=[
            pl.BlockSpec((_BLOCK, _DIM), lambda i: (i, 0)),
            pl.BlockSpec((_DIM, _DIM), lambda i: (0, 0)),
            pl.BlockSpec((1, _DIM), lambda i: (0, 0)),
            pl.BlockSpec((_DIM, _DIM), lambda i: (0, 0)),
            pl.BlockSpec((1, _DIM), lambda i: (0, 0)),
            pl.BlockSpec((3, _DIM), lambda i: (0, 0)),
        ],
        out_specs=pl.BlockSpec((_BLOCK, _DIM), lambda i: (i, 0)),
        out_shape=jax.ShapeDtypeStruct((n, _DIM), jnp.float32),
    )(h, w1b, b1r, w2b, b2r, anchors)
    return out


# speedup vs baseline: 1.9926x; 1.0293x over previous
import functools

import jax
import jax.numpy as jnp
from jax.experimental import pallas as pl

_DIM = 1024
_NUM_LAYERS = 6
_BLOCK = 512
_NPAD = 8


def _collapse_block(h_ref, w1_ref, b1_ref, w2_ref, b2_ref, anch_ref, s_ref,
                    out_ref):
    h = h_ref[...]
    w1 = w1_ref[...]
    w2 = w2_ref[...]
    b1 = b1_ref[...]
    b2 = b2_ref[...]
    anch = anch_ref[...]
    strengths = s_ref[...]
    an2_raw = jnp.sum(anch * anch, axis=-1, keepdims=True)
    anchors = anch / jnp.maximum(jnp.sqrt(an2_raw), 1e-12)
    a2 = jnp.sum(anchors * anchors, axis=-1)[None, :]
    anchors_b = anchors.astype(jnp.bfloat16)

    for _ in range(_NUM_LAYERS):
        hs = jnp.sum(h * h, axis=-1, keepdims=True)
        hb = h.astype(jnp.bfloat16)
        hidden = jnp.tanh(
            jax.lax.dot_general(hb, w1, (((1,), (0,)), ((), ())),
                                preferred_element_type=jnp.float32) + b1)
        delta = jax.lax.dot_general(hidden.astype(jnp.bfloat16), w2,
                                    (((1,), (0,)), ((), ())),
                                    preferred_element_type=jnp.float32) + b2
        dots = jax.lax.dot_general(hb, anchors_b, (((1,), (1,)), ((), ())),
                                   preferred_element_type=jnp.float32)
        hnorm = jnp.sqrt(hs)
        align = dots / jnp.maximum(hnorm, 1e-12)
        dn2 = jnp.maximum(hs - 2.0 * dots + a2, 0.0)
        dnorm = jnp.sqrt(dn2)
        c = strengths * (1.0 - align) / jnp.maximum(dnorm, 1e-12)
        big_c = jnp.sum(c, axis=-1, keepdims=True)
        fcorr = jax.lax.dot_general(c, anchors, (((1,), (0,)), ((), ())),
                                    preferred_element_type=jnp.float32)
        h = h * (1.0 - big_c) + delta + fcorr
        hs2 = jnp.sqrt(jnp.sum(h * h, axis=-1, keepdims=True))
        h = jnp.where(hs2 > 10.0, h * (10.0 / (hs2 + 1e-8)), h)
    out_ref[...] = h


@functools.partial(jax.jit, static_argnames=())
def kernel(h0, W1, b1, W2, b2, anchor_entail, anchor_contra, anchor_neutral):
    h = h0
    if h.ndim == 1:
        h = h[None, :]
    n = h.shape[0]
    anchors = jnp.zeros((_NPAD, _DIM), jnp.float32).at[:3].set(
        jnp.stack([anchor_entail, anchor_contra, anchor_neutral]))
    strengths = jnp.array([[0.1, 0.1, 0.05, 0.0, 0.0, 0.0, 0.0, 0.0]],
                          jnp.float32)
    w1t = W1.T.astype(jnp.bfloat16)
    w2t = W2.T.astype(jnp.bfloat16)
    b1r = b1.reshape(1, _DIM)
    b2r = b2.reshape(1, _DIM)
    grid = (n // _BLOCK,)
    out = pl.pallas_call(
        _collapse_block,
        grid=grid,
        in_specs=[
            pl.BlockSpec((_BLOCK, _DIM), lambda i: (i, 0)),
            pl.BlockSpec((_DIM, _DIM), lambda i: (0, 0)),
            pl.BlockSpec((1, _DIM), lambda i: (0, 0)),
            pl.BlockSpec((_DIM, _DIM), lambda i: (0, 0)),
            pl.BlockSpec((1, _DIM), lambda i: (0, 0)),
            pl.BlockSpec((_NPAD, _DIM), lambda i: (0, 0)),
            pl.BlockSpec((1, _NPAD), lambda i: (0, 0)),
        ],
        out_specs=pl.BlockSpec((_BLOCK, _DIM), lambda i: (i, 0)),
        out_shape=jax.ShapeDtypeStruct((n, _DIM), jnp.float32),
    )(h, w1t, b1r, w2t, b2r, anchors, strengths)
    return out


# speedup vs baseline: 2.3033x; 1.1559x over previous
import functools

import jax
import jax.numpy as jnp
from jax.experimental import pallas as pl

_DIM = 1024
_NUM_LAYERS = 6
_BLOCK = 512
_HALF = _BLOCK // 2
_NPAD = 8


def _collapse_block(h_ref, w1_ref, b1_ref, w2_ref, b2_ref, anch_ref, s_ref,
                    out_ref):
    w1 = w1_ref[...]
    w2 = w2_ref[...]
    b1 = b1_ref[...]
    b2 = b2_ref[...]
    anch = anch_ref[...]
    strengths = s_ref[...]
    an2_raw = jnp.sum(anch * anch, axis=-1, keepdims=True)
    anchors = anch / jnp.maximum(jnp.sqrt(an2_raw), 1e-12)
    a2 = jnp.sum(anchors * anchors, axis=-1)[None, :]
    anchors_b = anchors.astype(jnp.bfloat16)

    def step(g, s):
        hs = jnp.sum(g * g, axis=-1, keepdims=True) * (s * s)
        gb = (g * s).astype(jnp.bfloat16)
        hidden = jnp.tanh(
            jax.lax.dot_general(gb, w1, (((1,), (0,)), ((), ())),
                                preferred_element_type=jnp.float32) + b1)
        delta = jax.lax.dot_general(hidden.astype(jnp.bfloat16), w2,
                                    (((1,), (0,)), ((), ())),
                                    preferred_element_type=jnp.float32) + b2
        dots = jax.lax.dot_general(gb, anchors_b, (((1,), (1,)), ((), ())),
                                   preferred_element_type=jnp.float32)
        hnorm = jnp.sqrt(hs)
        align = dots / jnp.maximum(hnorm, 1e-12)
        dn2 = jnp.maximum(hs - 2.0 * dots + a2, 0.0)
        c = strengths * (1.0 - align) / jnp.maximum(jnp.sqrt(dn2), 1e-12)
        big_c = jnp.sum(c, axis=-1, keepdims=True)
        fcorr = jax.lax.dot_general(c, anchors, (((1,), (0,)), ((), ())),
                                    preferred_element_type=jnp.float32)
        g_new = g * (s * (1.0 - big_c)) + delta + fcorr
        n = jnp.sqrt(jnp.sum(g_new * g_new, axis=-1, keepdims=True))
        s_new = jnp.where(n > 10.0, 10.0 / (n + 1e-8), 1.0)
        return g_new, s_new

    ga = h_ref[0:_HALF, :]
    gb_ = h_ref[_HALF:_BLOCK, :]
    sa = jnp.ones((_HALF, 1), jnp.float32)
    sb = jnp.ones((_HALF, 1), jnp.float32)
    for _ in range(_NUM_LAYERS):
        ga, sa = step(ga, sa)
        gb_, sb = step(gb_, sb)
    out_ref[0:_HALF, :] = ga * sa
    out_ref[_HALF:_BLOCK, :] = gb_ * sb


@functools.partial(jax.jit, static_argnames=())
def kernel(h0, W1, b1, W2, b2, anchor_entail, anchor_contra, anchor_neutral):
    h = h0
    if h.ndim == 1:
        h = h[None, :]
    n = h.shape[0]
    anchors = jnp.zeros((_NPAD, _DIM), jnp.float32).at[:3].set(
        jnp.stack([anchor_entail, anchor_contra, anchor_neutral]))
    strengths = jnp.array([[0.1, 0.1, 0.05, 0.0, 0.0, 0.0, 0.0, 0.0]],
                          jnp.float32)
    w1t = W1.T.astype(jnp.bfloat16)
    w2t = W2.T.astype(jnp.bfloat16)
    b1r = b1.reshape(1, _DIM)
    b2r = b2.reshape(1, _DIM)
    grid = (n // _BLOCK,)
    out = pl.pallas_call(
        _collapse_block,
        grid=grid,
        in_specs=[
            pl.BlockSpec((_BLOCK, _DIM), lambda i: (i, 0)),
            pl.BlockSpec((_DIM, _DIM), lambda i: (0, 0)),
            pl.BlockSpec((1, _DIM), lambda i: (0, 0)),
            pl.BlockSpec((_DIM, _DIM), lambda i: (0, 0)),
            pl.BlockSpec((1, _DIM), lambda i: (0, 0)),
            pl.BlockSpec((_NPAD, _DIM), lambda i: (0, 0)),
            pl.BlockSpec((1, _NPAD), lambda i: (0, 0)),
        ],
        out_specs=pl.BlockSpec((_BLOCK, _DIM), lambda i: (i, 0)),
        out_shape=jax.ShapeDtypeStruct((n, _DIM), jnp.float32),
    )(h, w1t, b1r, w2t, b2r, anchors, strengths)
    return out
